# baseline (speedup 1.0000x reference)
import jax
import jax.numpy as jnp
from jax import lax
from jax.experimental import pallas as pl
from jax.experimental.pallas import tpu as pltpu

GS = 8
TOPK = 3


def _pool_body(xp_ref, d_ref):
    v = xp_ref[0]
    gw = v.shape[1] // GS
    c = v.shape[2]
    m = jnp.max(v, axis=0)
    d_ref[...] = jnp.max(m.reshape(gw, GS, c), axis=1)


def _topk_body(d_rows_ref, d_all_ref, val_ref, idx_ref):
    s = lax.dot_general(d_rows_ref[...], d_all_ref[...],
                        (((1,), (1,)), ((), ())),
                        preferred_element_type=jnp.float32)
    iota = lax.broadcasted_iota(jnp.int32, s.shape, 1)
    vals, idxs = [], []
    for _ in range(TOPK):
        v = jnp.max(s, axis=1, keepdims=True)
        i = jnp.min(jnp.where(s >= v, iota, jnp.int32(2 ** 30)),
                    axis=1, keepdims=True)
        vals.append(v)
        idxs.append(i)
        s = jnp.where(iota == i, -jnp.inf, s)
    r = s.shape[0]
    val_ref[...] = jnp.concatenate(
        vals + [jnp.full((r, 8 - TOPK), -1e30, jnp.float32)], axis=1)
    idx_ref[...] = jnp.concatenate(
        idxs + [jnp.zeros((r, 8 - TOPK), jnp.int32)], axis=1)


def _attn_body(*refs):
    (sidx_ref, xs_ref), rest = refs[:2], refs[2:]
    wb = (len(rest) - 4) // TOPK
    nrefs = rest[:TOPK * wb]
    sv_ref, wcat_ref, misc_ref, out_ref = rest[TOPK * wb:]
    p_dim = GS * GS
    c_dim = xs_ref.shape[-1]
    rows = wb * p_dim

    xv = xs_ref[0]
    xs = jnp.concatenate(
        [xv[:, j * GS:(j + 1) * GS, :].reshape(p_dim, c_dim)
         for j in range(wb)], axis=0)

    sv = sv_ref[...].reshape(wb, 8)
    lane = lax.broadcasted_iota(jnp.int32, sv.shape, 1)
    m3 = jnp.max(jnp.where(lane < TOPK, sv, -jnp.inf), axis=1, keepdims=True)
    e = jnp.where(lane < TOPK, jnp.exp(sv - m3), 0.0)
    w = e / jnp.sum(e, axis=1, keepdims=True)

    ctx = jnp.concatenate([
        sum(w[j:j + 1, k:k + 1]
            * nrefs[TOPK * j + k][0].reshape(p_dim, c_dim)
            for k in range(TOPK))
        for j in range(wb)], axis=0)

    def mm(a, b, dims):
        return lax.dot_general(a, b, (dims, ((), ())),
                               preferred_element_type=jnp.float32)

    wcat = wcat_ref[...]
    misc = misc_ref[...]
    q = mm(xs, wcat[0:c_dim], ((1,), (1,))) + misc[0:1]
    kk = mm(ctx, wcat[c_dim:2 * c_dim], ((1,), (1,))) + misc[1:2]
    vv = mm(ctx, wcat[2 * c_dim:3 * c_dim], ((1,), (1,))) + misc[2:3]

    pvec = lax.broadcasted_iota(jnp.int32, (rows, 1), 0)
    yy = (pvec % p_dim) // GS
    xx = pvec % GS
    acc = jnp.zeros((rows, c_dim), jnp.float32)
    for ky in range(3):
        for kx in range(3):
            dy, dx = ky - 1, kx - 1
            s = GS * dy + dx
            sh = xs if s == 0 else jnp.roll(xs, -s, axis=0)
            valid = ((yy + dy >= 0) & (yy + dy < GS)
                     & (xx + dx >= 0) & (xx + dx < GS))
            t = 3 * ky + kx
            acc = acc + jnp.where(valid, sh, 0.0) * misc[4 + t:5 + t, :]
    lepe = acc + misc[13:14]

    scale = c_dim ** -0.5
    attn = jnp.concatenate(
        [mm(q[j * p_dim:(j + 1) * p_dim], kk[j * p_dim:(j + 1) * p_dim],
            ((1,), (1,))) for j in range(wb)], axis=0) * scale
    mrow = jnp.max(attn, axis=1, keepdims=True)
    pexp = jnp.exp(attn - mrow)
    pnorm = pexp / jnp.sum(pexp, axis=1, keepdims=True)
    o = jnp.concatenate(
        [mm(pnorm[j * p_dim:(j + 1) * p_dim], vv[j * p_dim:(j + 1) * p_dim],
            ((1,), (0,))) for j in range(wb)], axis=0) + lepe
    res = mm(o, wcat[3 * c_dim:4 * c_dim], ((1,), (1,))) + misc[3:4]
    for j in range(wb):
        out_ref[0, :, j * GS:(j + 1) * GS, :] = (
            res[j * p_dim:(j + 1) * p_dim].reshape(GS, GS, c_dim))


def kernel(x, Wq, bq, Wk, bk, Wv, bv, Wp, bp, lepe_w, lepe_b):
    b, c, h, w = x.shape
    gh, gw = h // GS, w // GS
    n = b * gh * gw

    xp = jnp.transpose(x, (0, 2, 3, 1))

    d = pl.pallas_call(
        _pool_body,
        grid=(gh,),
        in_specs=[pl.BlockSpec((1, GS, w, c), lambda i: (0, i, 0, 0))],
        out_specs=pl.BlockSpec((gw, c), lambda i: (i, 0)),
        out_shape=jax.ShapeDtypeStruct((n, c), jnp.float32),
    )(xp)

    r = min(256, n)
    sval, sidx = pl.pallas_call(
        _topk_body,
        grid=(n // r,),
        in_specs=[pl.BlockSpec((r, c), lambda i: (i, 0)),
                  pl.BlockSpec((n, c), lambda i: (0, 0))],
        out_specs=[pl.BlockSpec((r, 8), lambda i: (i, 0)),
                   pl.BlockSpec((r, 8), lambda i: (i, 0))],
        out_shape=[jax.ShapeDtypeStruct((n, 8), jnp.float32),
                   jax.ShapeDtypeStruct((n, 8), jnp.int32)],
    )(d, d)

    sidx_flat = sidx[:, :TOPK].reshape(-1)
    sval3 = sval.reshape(n, 1, 8)
    lt = lepe_w.reshape(c, 9).transpose(1, 0)
    wcat = jnp.concatenate([Wq, Wk, Wv, Wp], axis=0)
    misc = jnp.concatenate(
        [bq.reshape(1, c), bk.reshape(1, c), bv.reshape(1, c),
         bp.reshape(1, c), lt, lepe_b.reshape(1, c),
         jnp.zeros((2, c), jnp.float32)], axis=0)

    wb = gw
    nspec = lambda t: pl.BlockSpec(
        (1, GS, GS, c),
        lambda i, si, t=t: (0, si[TOPK * wb * i + t] // gw,
                            si[TOPK * wb * i + t] % gw, 0))
    grid_spec = pltpu.PrefetchScalarGridSpec(
        num_scalar_prefetch=1,
        grid=(gh,),
        in_specs=[
            pl.BlockSpec((1, GS, w, c), lambda i, si: (0, i, 0, 0)),
            *[nspec(t) for t in range(TOPK * wb)],
            pl.BlockSpec((wb, 1, 8), lambda i, si: (i, 0, 0)),
            pl.BlockSpec((4 * c, c), lambda i, si: (0, 0)),
            pl.BlockSpec((16, c), lambda i, si: (0, 0)),
        ],
        out_specs=pl.BlockSpec((1, GS, w, c), lambda i, si: (0, i, 0, 0)),
    )
    out_xp = pl.pallas_call(
        _attn_body,
        grid_spec=grid_spec,
        out_shape=jax.ShapeDtypeStruct((b, h, w, c), jnp.float32),
    )(sidx_flat, xp, *([xp] * (TOPK * wb)), sval3, wcat, misc)

    return jnp.transpose(out_xp, (0, 3, 1, 2))


# speedup vs baseline: 1.1716x; 1.1716x over previous
import jax
import jax.numpy as jnp
from jax import lax
from jax.experimental import pallas as pl
from jax.experimental.pallas import tpu as pltpu

GS = 8
TOPK = 3


def _pool_body(xp_ref, xw_ref, d_ref):
    xv = xp_ref[0]
    gw = xv.shape[1] // GS
    c = xv.shape[2]
    p = GS * GS
    xs = jnp.concatenate(
        [xv[:, j * GS:(j + 1) * GS, :].reshape(p, c) for j in range(gw)],
        axis=0).reshape(gw, p, c)
    xw_ref[...] = xs
    d_ref[...] = jnp.max(xs, axis=1)


def _topk_body(d_rows_ref, d_all_ref, val_ref, idx_ref):
    s = lax.dot_general(d_rows_ref[...], d_all_ref[...],
                        (((1,), (1,)), ((), ())),
                        preferred_element_type=jnp.float32)
    iota = lax.broadcasted_iota(jnp.int32, s.shape, 1)
    vals, idxs = [], []
    for _ in range(TOPK):
        v = jnp.max(s, axis=1, keepdims=True)
        i = jnp.min(jnp.where(s >= v, iota, jnp.int32(2 ** 30)),
                    axis=1, keepdims=True)
        vals.append(v)
        idxs.append(i)
        s = jnp.where(iota == i, -jnp.inf, s)
    r = s.shape[0]
    val_ref[...] = jnp.concatenate(
        vals + [jnp.full((r, 8 - TOPK), -1e30, jnp.float32)], axis=1)
    idx_ref[...] = jnp.concatenate(
        idxs + [jnp.zeros((r, 8 - TOPK), jnp.int32)], axis=1)


def _attn_body(*refs):
    (sidx_ref, xs_ref), rest = refs[:2], refs[2:]
    wb = (len(rest) - 4) // TOPK
    nrefs = rest[:TOPK * wb]
    sv_ref, wcat_ref, misc_ref, out_ref = rest[TOPK * wb:]
    p_dim = GS * GS
    c_dim = xs_ref.shape[-1]
    rows = wb * p_dim

    xs = xs_ref[...].reshape(rows, c_dim)

    sv = sv_ref[...].reshape(wb, 8)
    lane = lax.broadcasted_iota(jnp.int32, sv.shape, 1)
    m3 = jnp.max(jnp.where(lane < TOPK, sv, -jnp.inf), axis=1, keepdims=True)
    e = jnp.where(lane < TOPK, jnp.exp(sv - m3), 0.0)
    w = e / jnp.sum(e, axis=1, keepdims=True)

    ctx = jnp.concatenate([
        sum(w[j:j + 1, k:k + 1] * nrefs[TOPK * j + k][0] for k in range(TOPK))
        for j in range(wb)], axis=0)

    def mm(a, b, dims):
        return lax.dot_general(a, b, (dims, ((), ())),
                               preferred_element_type=jnp.float32)

    wcat = wcat_ref[...]
    misc = misc_ref[...]
    q = mm(xs, wcat[0:c_dim], ((1,), (1,))) + misc[0:1]
    kk = mm(ctx, wcat[c_dim:2 * c_dim], ((1,), (1,))) + misc[1:2]
    vv = mm(ctx, wcat[2 * c_dim:3 * c_dim], ((1,), (1,))) + misc[2:3]

    pvec = lax.broadcasted_iota(jnp.int32, (rows, 1), 0)
    yy = (pvec % p_dim) // GS
    xx = pvec % GS
    acc = jnp.zeros((rows, c_dim), jnp.float32)
    for ky in range(3):
        for kx in range(3):
            dy, dx = ky - 1, kx - 1
            s = GS * dy + dx
            sh = xs if s == 0 else jnp.roll(xs, -s, axis=0)
            valid = ((yy + dy >= 0) & (yy + dy < GS)
                     & (xx + dx >= 0) & (xx + dx < GS))
            t = 3 * ky + kx
            acc = acc + jnp.where(valid, sh, 0.0) * misc[4 + t:5 + t, :]
    lepe = acc + misc[13:14]

    scale = c_dim ** -0.5
    attn = jnp.concatenate(
        [mm(q[j * p_dim:(j + 1) * p_dim], kk[j * p_dim:(j + 1) * p_dim],
            ((1,), (1,))) for j in range(wb)], axis=0) * scale
    mrow = jnp.max(attn, axis=1, keepdims=True)
    pexp = jnp.exp(attn - mrow)
    pnorm = pexp / jnp.sum(pexp, axis=1, keepdims=True)
    o = jnp.concatenate(
        [mm(pnorm[j * p_dim:(j + 1) * p_dim], vv[j * p_dim:(j + 1) * p_dim],
            ((1,), (0,))) for j in range(wb)], axis=0) + lepe
    res = mm(o, wcat[3 * c_dim:4 * c_dim], ((1,), (1,))) + misc[3:4]
    out_ref[0] = jnp.concatenate(
        [res[j * p_dim:(j + 1) * p_dim].reshape(GS, GS, c_dim)
         for j in range(wb)], axis=1)


def kernel(x, Wq, bq, Wk, bk, Wv, bv, Wp, bp, lepe_w, lepe_b):
    b, c, h, w = x.shape
    gh, gw = h // GS, w // GS
    n = b * gh * gw
    p = GS * GS

    xp = jnp.transpose(x, (0, 2, 3, 1))

    xw, d = pl.pallas_call(
        _pool_body,
        grid=(gh,),
        in_specs=[pl.BlockSpec((1, GS, w, c), lambda i: (0, i, 0, 0))],
        out_specs=[pl.BlockSpec((gw, p, c), lambda i: (i, 0, 0)),
                   pl.BlockSpec((gw, c), lambda i: (i, 0))],
        out_shape=[jax.ShapeDtypeStruct((n, p, c), jnp.float32),
                   jax.ShapeDtypeStruct((n, c), jnp.float32)],
    )(xp)

    r = min(256, n)
    sval, sidx = pl.pallas_call(
        _topk_body,
        grid=(n // r,),
        in_specs=[pl.BlockSpec((r, c), lambda i: (i, 0)),
                  pl.BlockSpec((n, c), lambda i: (0, 0))],
        out_specs=[pl.BlockSpec((r, 8), lambda i: (i, 0)),
                   pl.BlockSpec((r, 8), lambda i: (i, 0))],
        out_shape=[jax.ShapeDtypeStruct((n, 8), jnp.float32),
                   jax.ShapeDtypeStruct((n, 8), jnp.int32)],
    )(d, d)

    sidx_flat = sidx[:, :TOPK].reshape(-1)
    sval3 = sval.reshape(n, 1, 8)
    lt = lepe_w.reshape(c, 9).transpose(1, 0)
    wcat = jnp.concatenate([Wq, Wk, Wv, Wp], axis=0)
    misc = jnp.concatenate(
        [bq.reshape(1, c), bk.reshape(1, c), bv.reshape(1, c),
         bp.reshape(1, c), lt, lepe_b.reshape(1, c),
         jnp.zeros((2, c), jnp.float32)], axis=0)

    wb = gw
    nspec = lambda t: pl.BlockSpec(
        (1, p, c), lambda i, si, t=t: (si[TOPK * wb * i + t], 0, 0))
    grid_spec = pltpu.PrefetchScalarGridSpec(
        num_scalar_prefetch=1,
        grid=(gh,),
        in_specs=[
            pl.BlockSpec((wb, p, c), lambda i, si: (i, 0, 0)),
            *[nspec(t) for t in range(TOPK * wb)],
            pl.BlockSpec((wb, 1, 8), lambda i, si: (i, 0, 0)),
            pl.BlockSpec((4 * c, c), lambda i, si: (0, 0)),
            pl.BlockSpec((16, c), lambda i, si: (0, 0)),
        ],
        out_specs=pl.BlockSpec((1, GS, w, c), lambda i, si: (0, i, 0, 0)),
    )
    out_xp = pl.pallas_call(
        _attn_body,
        grid_spec=grid_spec,
        out_shape=jax.ShapeDtypeStruct((b, h, w, c), jnp.float32),
    )(sidx_flat, xw, *([xw] * (TOPK * wb)), sval3, wcat, misc)

    return jnp.transpose(out_xp, (0, 3, 1, 2))
